# baseline (speedup 1.0000x reference)
import functools

import jax
import jax.numpy as jnp
from jax import lax
from jax.experimental import pallas as pl
from jax.experimental.pallas import tpu as pltpu
from jax.experimental.pallas import tpu_sc as plsc

TOP_K = 2
BLK = 128
TB = 512
NUM_CORES = 2
NUM_SUBCORES = 16
NW = NUM_CORES * NUM_SUBCORES


def _router_body(x_ref, gw_ref, i0_ref, i1_ref, p0_ref, p1_ref):
    xb = x_ref[...]
    gw = gw_ref[...]
    s = lax.dot_general(xb, gw, (((1,), (1,)), ((), ())),
                        preferred_element_type=jnp.float32)
    e = s.shape[1]
    cols = lax.broadcasted_iota(jnp.int32, s.shape, 1)
    m0 = jnp.max(s, axis=1)
    i0 = jnp.min(jnp.where(s == m0[:, None], cols, e), axis=1)
    s2 = jnp.where(cols == i0[:, None], -jnp.inf, s)
    m1 = jnp.max(s2, axis=1)
    i1 = jnp.min(jnp.where(s2 == m1[:, None], cols, e), axis=1)
    p0 = 1.0 / (1.0 + jnp.exp(m1 - m0))
    i0_ref[...] = i0[:, None].astype(jnp.int32)
    i1_ref[...] = i1[:, None].astype(jnp.int32)
    p0_ref[...] = p0[:, None]
    p1_ref[...] = (1.0 - p0)[:, None]


def _router(x2d, gate_w):
    t, d = x2d.shape
    e = gate_w.shape[0]
    outs = pl.pallas_call(
        _router_body,
        grid=(t // TB,),
        in_specs=[
            pl.BlockSpec((TB, d), lambda b: (b, 0)),
            pl.BlockSpec((e, d), lambda b: (0, 0)),
        ],
        out_specs=[pl.BlockSpec((TB, 1), lambda b: (b, 0))] * 4,
        out_shape=[
            jax.ShapeDtypeStruct((t, 1), jnp.int32),
            jax.ShapeDtypeStruct((t, 1), jnp.int32),
            jax.ShapeDtypeStruct((t, 1), jnp.float32),
            jax.ShapeDtypeStruct((t, 1), jnp.float32),
        ],
    )(x2d, gate_w)
    return [o[:, 0] for o in outs]


def _sc_bookkeep(pos, p, t):
    nslots = pos.shape[0]
    spw = nslots // NW
    nj = spw // 128
    mesh = plsc.VectorSubcoreMesh(core_axis_name="c", subcore_axis_name="s")
    pos3 = pos.reshape(NW, nj, 128)
    tok3 = jnp.broadcast_to(
        (jnp.arange(nslots, dtype=jnp.int32) >> 1).reshape(NW, nj, 128, 1),
        (NW, nj, 128, 128))

    @functools.partial(
        pl.kernel,
        out_type=jax.ShapeDtypeStruct((p, 128), jnp.int32),
        mesh=mesh,
        scratch_types=[
            pltpu.VMEM((nj, 128), jnp.int32),
            pltpu.VMEM((nj, 128, 128), jnp.int32),
            pltpu.SemaphoreType.DMA,
            pltpu.SemaphoreType.DMA,
        ],
    )
    def bk_k(pos_hbm, tok_hbm, out_hbm, posw, tokw, sem0, sem1):
        wid = lax.axis_index("s") * NUM_CORES + lax.axis_index("c")
        pltpu.sync_copy(pos_hbm.at[wid], posw)
        pltpu.sync_copy(tok_hbm.at[wid], tokw)
        sems = (sem0, sem1)
        cps = [pltpu.async_copy(tokw.at[j], out_hbm.at[posw.at[j]], sems[j])
               for j in range(nj)]
        for cp in cps:
            cp.wait()

    return bk_k(pos3, tok3)[:, 0]


def _sc_gather(x2d, row_token, p):
    t = x2d.shape[0]
    row = x2d.shape[1:]
    rpw = p // NW
    ch = 8
    nbuf = 6
    n = rpw // ch
    ilen = ((rpw + 15) // 16) * 16
    mesh = plsc.VectorSubcoreMesh(core_axis_name="c", subcore_axis_name="s")

    @functools.partial(
        pl.kernel,
        out_type=jax.ShapeDtypeStruct((p,) + row, x2d.dtype),
        mesh=mesh,
        scratch_types=[pltpu.VMEM((ilen,), jnp.int32)]
        + [pltpu.VMEM((ch,) + row, x2d.dtype)] * nbuf
        + [pltpu.SemaphoreType.DMA] * (2 * nbuf),
    )
    def gather_k(x_hbm, tok_hbm, out_hbm, idx_all, *bufs_sems):
        rows = bufs_sems[:nbuf]
        sg = bufs_sems[nbuf:2 * nbuf]
        ss = bufs_sems[2 * nbuf:]
        wid = lax.axis_index("s") * NUM_CORES + lax.axis_index("c")
        base = wid * rpw
        pltpu.sync_copy(tok_hbm.at[pl.ds(base, rpw)],
                        idx_all.at[pl.ds(0, rpw)])

        def clamp(i, carry):
            sl = pl.ds(i * 16, 16)
            v = idx_all[sl]
            idx_all[sl] = jnp.minimum(jnp.maximum(v, 0), t - 1)
            return carry

        lax.fori_loop(0, ilen // 16, clamp, 0)
        gat = [None] * nbuf
        st = [None] * nbuf

        def start(i):
            b = i % nbuf
            gat[b] = pltpu.async_copy(
                x_hbm.at[idx_all.at[pl.ds(i * ch, ch)]], rows[b], sg[b])

        for i in range(min(nbuf, n)):
            start(i)
        for i in range(n):
            b = i % nbuf
            gat[b].wait()
            st[b] = pltpu.async_copy(
                rows[b], out_hbm.at[pl.ds(base + i * ch, ch)], ss[b])
            j = i + nbuf
            if j < n:
                st[b].wait()
                start(j)
        for i in range(max(0, n - nbuf), n):
            st[i % nbuf].wait()

    return gather_k(x2d, row_token)


def _sc_combine(ys, pos0, pos1, p0b, p1b):
    p, d = ys.shape
    t = pos0.shape[0]
    tpw = t // NW
    ch = 8
    nbuf = 2
    n = tpw // ch
    mesh = plsc.VectorSubcoreMesh(core_axis_name="c", subcore_axis_name="s")

    @functools.partial(
        pl.kernel,
        out_type=jax.ShapeDtypeStruct((t, d), jnp.float32),
        mesh=mesh,
        scratch_types=[pltpu.VMEM((tpw,), jnp.int32),
                       pltpu.VMEM((tpw,), jnp.int32),
                       pltpu.VMEM((tpw, 16), jnp.float32),
                       pltpu.VMEM((tpw, 16), jnp.float32)]
        + [pltpu.VMEM((ch, d), jnp.float32)] * (2 * nbuf)
        + [pltpu.SemaphoreType.DMA] * (3 * nbuf),
    )
    def combine_k(y_hbm, p0_hbm, p1_hbm, s0_hbm, s1_hbm, out_hbm,
                  i0_all, i1_all, s0_all, s1_all, *bufs_sems):
        av = bufs_sems[:nbuf]
        bv = bufs_sems[nbuf:2 * nbuf]
        sems = bufs_sems[2 * nbuf:]
        sa = sems[:nbuf]
        sb = sems[nbuf:2 * nbuf]
        ss = sems[2 * nbuf:]
        wid = lax.axis_index("s") * NUM_CORES + lax.axis_index("c")
        base = wid * tpw
        pltpu.sync_copy(p0_hbm.at[pl.ds(base, tpw)], i0_all)
        pltpu.sync_copy(p1_hbm.at[pl.ds(base, tpw)], i1_all)
        pltpu.sync_copy(s0_hbm.at[pl.ds(base, tpw)], s0_all)
        pltpu.sync_copy(s1_hbm.at[pl.ds(base, tpw)], s1_all)
        ga = [None] * nbuf
        gb = [None] * nbuf
        st = [None] * nbuf

        def start(i):
            k = i % nbuf
            sl = pl.ds(i * ch, ch)
            ga[k] = pltpu.async_copy(y_hbm.at[i0_all.at[sl]], av[k], sa[k])
            gb[k] = pltpu.async_copy(y_hbm.at[i1_all.at[sl]], bv[k], sb[k])

        for i in range(min(nbuf, n)):
            start(i)
        for i in range(n):
            k = i % nbuf
            ga[k].wait()
            gb[k].wait()
            a_ref, b_ref = av[k], bv[k]
            s0r = [s0_all[i * ch + r, :] for r in range(ch)]
            s1r = [s1_all[i * ch + r, :] for r in range(ch)]

            def add_col(c, carry, a_ref=a_ref, b_ref=b_ref,
                        s0r=s0r, s1r=s1r):
                sl = pl.ds(c * 16, 16)
                for r in range(ch):
                    a_ref[r, sl] = (s0r[r] * a_ref[r, sl]
                                    + s1r[r] * b_ref[r, sl])
                return carry

            lax.fori_loop(0, d // 16, add_col, 0)
            st[k] = pltpu.async_copy(
                a_ref, out_hbm.at[pl.ds(base + i * ch, ch)], ss[k])
            j = i + nbuf
            if j < n:
                st[k].wait()
                start(j)
        for i in range(max(0, n - nbuf), n):
            st[i % nbuf].wait()

    return combine_k(ys, pos0, pos1, p0b, p1b)


def _mlp1_body(be_ref, xs_ref, w1_ref, w2_ref, h_ref):
    xb = xs_ref[...].astype(jnp.bfloat16)
    w1 = w1_ref[0].astype(jnp.bfloat16)
    w2 = w2_ref[0].astype(jnp.bfloat16)
    dn = (((1,), (1,)), ((), ()))
    h1 = lax.dot_general(xb, w1, dn, preferred_element_type=jnp.float32)
    h2 = lax.dot_general(xb, w2, dn, preferred_element_type=jnp.float32)
    h_ref[...] = (h1 * jax.nn.sigmoid(h1) * h2).astype(jnp.bfloat16)


def _mlp2_body(be_ref, h_ref, w3_ref, out_ref):
    hb = h_ref[...]
    w3 = w3_ref[0].astype(jnp.bfloat16)
    y = lax.dot_general(hb, w3, (((1,), (1,)), ((), ())),
                        preferred_element_type=jnp.float32)
    out_ref[...] = y


def _mlp2_body_alias(be_ref, h_ref, w3_ref, ys_ref, out_ref):
    del ys_ref
    _mlp2_body(be_ref, h_ref, w3_ref, out_ref)


def _mlp1(be_q, xs_q, fc1_w, fc2_w):
    pq, d = xs_q.shape
    e, f, _ = fc1_w.shape
    return pl.pallas_call(
        _mlp1_body,
        grid_spec=pltpu.PrefetchScalarGridSpec(
            num_scalar_prefetch=1,
            grid=(pq // BLK,),
            in_specs=[
                pl.BlockSpec((BLK, d), lambda b, be: (b, 0)),
                pl.BlockSpec((1, f, d), lambda b, be: (be[b], 0, 0)),
                pl.BlockSpec((1, f, d), lambda b, be: (be[b], 0, 0)),
            ],
            out_specs=pl.BlockSpec((BLK, f), lambda b, be: (b, 0)),
        ),
        out_shape=jax.ShapeDtypeStruct((pq, f), jnp.bfloat16),
    )(be_q, xs_q, fc1_w, fc2_w)


def _mlp2(be_q, hs_q, fc3_w, q0blk, p, ys_prev):
    pq, f = hs_q.shape
    d = fc3_w.shape[1]
    in_specs = [
        pl.BlockSpec((BLK, f), lambda b, be: (b, 0)),
        pl.BlockSpec((1, d, f), lambda b, be: (be[b], 0, 0)),
    ]
    operands = [be_q, hs_q, fc3_w]
    body = _mlp2_body
    aliases = {}
    if ys_prev is not None:
        in_specs.append(pl.BlockSpec(memory_space=pltpu.MemorySpace.HBM))
        operands.append(ys_prev)
        body = _mlp2_body_alias
        aliases = {3: 0}
    return pl.pallas_call(
        body,
        grid_spec=pltpu.PrefetchScalarGridSpec(
            num_scalar_prefetch=1,
            grid=(pq // BLK,),
            in_specs=in_specs,
            out_specs=pl.BlockSpec((BLK, d),
                                   lambda b, be, q0=q0blk: (q0 + b, 0)),
        ),
        out_shape=jax.ShapeDtypeStruct((p, d), jnp.float32),
        input_output_aliases=aliases,
    )(*operands)


def kernel(x, gate_w, fc1_w, fc2_w, fc3_w):
    b, s, d = x.shape
    e = gate_w.shape[0]
    t = b * s
    p = TOP_K * t + e * BLK
    x2d = x.reshape(t, d)

    i0, i1, p0, p1 = _router(x2d, gate_w)

    e_flat = jnp.stack([i0, i1], axis=1).reshape(-1)
    oh = (e_flat[:, None] == jnp.arange(e, dtype=jnp.int32)[None, :]).astype(
        jnp.int32)
    cum = jnp.cumsum(oh, axis=0)
    rank = jnp.sum((cum - oh) * oh, axis=1)
    counts = cum[-1]
    padded = ((counts + BLK - 1) // BLK) * BLK
    starts = jnp.concatenate(
        [jnp.zeros((1,), jnp.int32), jnp.cumsum(padded)[:-1].astype(jnp.int32)])
    pos = (jnp.sum(oh * starts[None, :], axis=1) + rank).astype(jnp.int32)
    pos2 = pos.reshape(t, TOP_K)
    blk_lo = jnp.arange(p // BLK, dtype=jnp.int32)[:, None] * BLK
    block_expert = jnp.sum((blk_lo >= starts[None, :]).astype(jnp.int32),
                           axis=1) - 1
    p0b = jnp.broadcast_to(p0[:, None], (t, 16))
    p1b = jnp.broadcast_to(p1[:, None], (t, 16))

    row_token = _sc_bookkeep(pos, p, t)

    nq = 4
    pq = p // nq
    nblk_q = pq // BLK
    xs_q = [_sc_gather(x2d, row_token[q * pq:(q + 1) * pq], pq)
            for q in range(nq)]
    ys = None
    for q in range(nq):
        be_q = block_expert[q * nblk_q:(q + 1) * nblk_q]
        hs_q = _mlp1(be_q, xs_q[q], fc1_w, fc2_w)
        ys = _mlp2(be_q, hs_q, fc3_w, q * nblk_q, p, ys)
    out2d = _sc_combine(ys, pos2[:, 0], pos2[:, 1], p0b, p1b)
    return out2d.reshape(b, s, d)


# speedup vs baseline: 1.2524x; 1.2524x over previous
import functools

import jax
import jax.numpy as jnp
from jax import lax
from jax.experimental import pallas as pl
from jax.experimental.pallas import tpu as pltpu
from jax.experimental.pallas import tpu_sc as plsc

TOP_K = 2
BLK = 256
TB = 512
NUM_CORES = 2
NUM_SUBCORES = 16
NW = NUM_CORES * NUM_SUBCORES


def _router_body(x_ref, gw_ref, i0_ref, i1_ref, p0_ref, p1_ref):
    xb = x_ref[...]
    gw = gw_ref[...]
    s = lax.dot_general(xb, gw, (((1,), (1,)), ((), ())),
                        preferred_element_type=jnp.float32)
    e = s.shape[1]
    cols = lax.broadcasted_iota(jnp.int32, s.shape, 1)
    m0 = jnp.max(s, axis=1)
    i0 = jnp.min(jnp.where(s == m0[:, None], cols, e), axis=1)
    s2 = jnp.where(cols == i0[:, None], -jnp.inf, s)
    m1 = jnp.max(s2, axis=1)
    i1 = jnp.min(jnp.where(s2 == m1[:, None], cols, e), axis=1)
    p0 = 1.0 / (1.0 + jnp.exp(m1 - m0))
    i0_ref[...] = i0[:, None].astype(jnp.int32)
    i1_ref[...] = i1[:, None].astype(jnp.int32)
    p0_ref[...] = p0[:, None]
    p1_ref[...] = (1.0 - p0)[:, None]


def _router(x2d, gate_w):
    t, d = x2d.shape
    e = gate_w.shape[0]
    outs = pl.pallas_call(
        _router_body,
        grid=(t // TB,),
        in_specs=[
            pl.BlockSpec((TB, d), lambda b: (b, 0)),
            pl.BlockSpec((e, d), lambda b: (0, 0)),
        ],
        out_specs=[pl.BlockSpec((TB, 1), lambda b: (b, 0))] * 4,
        out_shape=[
            jax.ShapeDtypeStruct((t, 1), jnp.int32),
            jax.ShapeDtypeStruct((t, 1), jnp.int32),
            jax.ShapeDtypeStruct((t, 1), jnp.float32),
            jax.ShapeDtypeStruct((t, 1), jnp.float32),
        ],
    )(x2d, gate_w)
    return [o[:, 0] for o in outs]


def _sc_bookkeep(pos, p, t):
    nslots = pos.shape[0]
    spw = nslots // NW
    nj = spw // 128
    mesh = plsc.VectorSubcoreMesh(core_axis_name="c", subcore_axis_name="s")
    pos3 = pos.reshape(NW, nj, 128)
    tok3 = jnp.broadcast_to(
        (jnp.arange(nslots, dtype=jnp.int32) >> 1).reshape(NW, nj, 128, 1),
        (NW, nj, 128, 128))

    @functools.partial(
        pl.kernel,
        out_type=jax.ShapeDtypeStruct((p, 128), jnp.int32),
        mesh=mesh,
        scratch_types=[
            pltpu.VMEM((nj, 128), jnp.int32),
            pltpu.VMEM((nj, 128, 128), jnp.int32),
            pltpu.SemaphoreType.DMA,
            pltpu.SemaphoreType.DMA,
        ],
    )
    def bk_k(pos_hbm, tok_hbm, out_hbm, posw, tokw, sem0, sem1):
        wid = lax.axis_index("s") * NUM_CORES + lax.axis_index("c")
        pltpu.sync_copy(pos_hbm.at[wid], posw)
        pltpu.sync_copy(tok_hbm.at[wid], tokw)
        sems = (sem0, sem1)
        cps = [pltpu.async_copy(tokw.at[j], out_hbm.at[posw.at[j]], sems[j])
               for j in range(nj)]
        for cp in cps:
            cp.wait()

    return bk_k(pos3, tok3)[:, 0]


def _sc_gather(x2d, row_token, p):
    t = x2d.shape[0]
    row = x2d.shape[1:]
    rpw = p // NW
    ch = 8
    nbuf = 6
    n = rpw // ch
    ilen = ((rpw + 15) // 16) * 16
    mesh = plsc.VectorSubcoreMesh(core_axis_name="c", subcore_axis_name="s")

    @functools.partial(
        pl.kernel,
        out_type=jax.ShapeDtypeStruct((p,) + row, x2d.dtype),
        mesh=mesh,
        scratch_types=[pltpu.VMEM((ilen,), jnp.int32)]
        + [pltpu.VMEM((ch,) + row, x2d.dtype)] * nbuf
        + [pltpu.SemaphoreType.DMA] * (2 * nbuf),
    )
    def gather_k(x_hbm, tok_hbm, out_hbm, idx_all, *bufs_sems):
        rows = bufs_sems[:nbuf]
        sg = bufs_sems[nbuf:2 * nbuf]
        ss = bufs_sems[2 * nbuf:]
        wid = lax.axis_index("s") * NUM_CORES + lax.axis_index("c")
        base = wid * rpw
        pltpu.sync_copy(tok_hbm.at[pl.ds(base, rpw)],
                        idx_all.at[pl.ds(0, rpw)])

        def clamp(i, carry):
            sl = pl.ds(i * 16, 16)
            v = idx_all[sl]
            idx_all[sl] = jnp.minimum(jnp.maximum(v, 0), t - 1)
            return carry

        lax.fori_loop(0, ilen // 16, clamp, 0)
        gat = [None] * nbuf
        st = [None] * nbuf

        def start(i):
            b = i % nbuf
            gat[b] = pltpu.async_copy(
                x_hbm.at[idx_all.at[pl.ds(i * ch, ch)]], rows[b], sg[b])

        for i in range(min(nbuf, n)):
            start(i)
        for i in range(n):
            b = i % nbuf
            gat[b].wait()
            st[b] = pltpu.async_copy(
                rows[b], out_hbm.at[pl.ds(base + i * ch, ch)], ss[b])
            j = i + nbuf
            if j < n:
                st[b].wait()
                start(j)
        for i in range(max(0, n - nbuf), n):
            st[i % nbuf].wait()

    return gather_k(x2d, row_token)


def _sc_combine(ys, pos0, pos1, p0b, p1b):
    p, d = ys.shape
    t = pos0.shape[0]
    tpw = t // NW
    ch = 8
    nbuf = 3
    n = tpw // ch
    mesh = plsc.VectorSubcoreMesh(core_axis_name="c", subcore_axis_name="s")

    @functools.partial(
        pl.kernel,
        out_type=jax.ShapeDtypeStruct((t, d), jnp.float32),
        mesh=mesh,
        scratch_types=[pltpu.VMEM((tpw,), jnp.int32),
                       pltpu.VMEM((tpw,), jnp.int32),
                       pltpu.VMEM((ch, 16), jnp.float32),
                       pltpu.VMEM((ch, 16), jnp.float32)]
        + [pltpu.VMEM((ch, d), jnp.float32)] * (2 * nbuf)
        + [pltpu.SemaphoreType.DMA] * (3 * nbuf),
    )
    def combine_k(y_hbm, p0_hbm, p1_hbm, s0_hbm, s1_hbm, out_hbm,
                  i0_all, i1_all, s0_all, s1_all, *bufs_sems):
        av = bufs_sems[:nbuf]
        bv = bufs_sems[nbuf:2 * nbuf]
        sems = bufs_sems[2 * nbuf:]
        sa = sems[:nbuf]
        sb = sems[nbuf:2 * nbuf]
        ss = sems[2 * nbuf:]
        wid = lax.axis_index("s") * NUM_CORES + lax.axis_index("c")
        base = wid * tpw
        pltpu.sync_copy(p0_hbm.at[pl.ds(base, tpw)], i0_all)
        pltpu.sync_copy(p1_hbm.at[pl.ds(base, tpw)], i1_all)
        ga = [None] * nbuf
        gb = [None] * nbuf
        st = [None] * nbuf

        def start(i):
            k = i % nbuf
            sl = pl.ds(i * ch, ch)
            ga[k] = pltpu.async_copy(y_hbm.at[i0_all.at[sl]], av[k], sa[k])
            gb[k] = pltpu.async_copy(y_hbm.at[i1_all.at[sl]], bv[k], sb[k])

        for i in range(min(nbuf, n)):
            start(i)
        for i in range(n):
            k = i % nbuf
            pltpu.sync_copy(s0_hbm.at[pl.ds(base + i * ch, ch)], s0_all)
            pltpu.sync_copy(s1_hbm.at[pl.ds(base + i * ch, ch)], s1_all)
            ga[k].wait()
            gb[k].wait()
            a_ref, b_ref = av[k], bv[k]
            s0r = [s0_all[r, :] for r in range(ch)]
            s1r = [s1_all[r, :] for r in range(ch)]

            def add_col(c, carry, a_ref=a_ref, b_ref=b_ref,
                        s0r=s0r, s1r=s1r):
                sl = pl.ds(c * 16, 16)
                for r in range(ch):
                    a_ref[r, sl] = (s0r[r] * a_ref[r, sl]
                                    + s1r[r] * b_ref[r, sl])
                return carry

            lax.fori_loop(0, d // 16, add_col, 0)
            st[k] = pltpu.async_copy(
                a_ref, out_hbm.at[pl.ds(base + i * ch, ch)], ss[k])
            j = i + nbuf
            if j < n:
                st[k].wait()
                start(j)
        for i in range(max(0, n - nbuf), n):
            st[i % nbuf].wait()

    return combine_k(ys, pos0, pos1, p0b, p1b)


def _mlp1_body(be_ref, xs_ref, w1_ref, w2_ref, h_ref):
    xb = xs_ref[...].astype(jnp.bfloat16)
    w1 = w1_ref[0].astype(jnp.bfloat16)
    w2 = w2_ref[0].astype(jnp.bfloat16)
    dn = (((1,), (1,)), ((), ()))
    h1 = lax.dot_general(xb, w1, dn, preferred_element_type=jnp.float32)
    h2 = lax.dot_general(xb, w2, dn, preferred_element_type=jnp.float32)
    h_ref[...] = (h1 * jax.nn.sigmoid(h1) * h2).astype(jnp.bfloat16)


def _mlp2_body(be_ref, h_ref, w3_ref, out_ref):
    hb = h_ref[...]
    w3 = w3_ref[0].astype(jnp.bfloat16)
    y = lax.dot_general(hb, w3, (((1,), (1,)), ((), ())),
                        preferred_element_type=jnp.float32)
    out_ref[...] = y


def _mlp2_body_alias(be_ref, h_ref, w3_ref, ys_ref, out_ref):
    del ys_ref
    _mlp2_body(be_ref, h_ref, w3_ref, out_ref)


def _mlp1(be_q, xs_q, fc1_w, fc2_w):
    pq, d = xs_q.shape
    e, f, _ = fc1_w.shape
    return pl.pallas_call(
        _mlp1_body,
        grid_spec=pltpu.PrefetchScalarGridSpec(
            num_scalar_prefetch=1,
            grid=(pq // BLK,),
            in_specs=[
                pl.BlockSpec((BLK, d), lambda b, be: (b, 0)),
                pl.BlockSpec((1, f, d), lambda b, be: (be[b], 0, 0)),
                pl.BlockSpec((1, f, d), lambda b, be: (be[b], 0, 0)),
            ],
            out_specs=pl.BlockSpec((BLK, f), lambda b, be: (b, 0)),
        ),
        out_shape=jax.ShapeDtypeStruct((pq, f), jnp.bfloat16),
    )(be_q, xs_q, fc1_w, fc2_w)


def _mlp2(be_q, hs_q, fc3_w, q0blk, p, ys_prev):
    pq, f = hs_q.shape
    d = fc3_w.shape[1]
    in_specs = [
        pl.BlockSpec((BLK, f), lambda b, be: (b, 0)),
        pl.BlockSpec((1, d, f), lambda b, be: (be[b], 0, 0)),
    ]
    operands = [be_q, hs_q, fc3_w]
    body = _mlp2_body
    aliases = {}
    if ys_prev is not None:
        in_specs.append(pl.BlockSpec(memory_space=pltpu.MemorySpace.HBM))
        operands.append(ys_prev)
        body = _mlp2_body_alias
        aliases = {3: 0}
    return pl.pallas_call(
        body,
        grid_spec=pltpu.PrefetchScalarGridSpec(
            num_scalar_prefetch=1,
            grid=(pq // BLK,),
            in_specs=in_specs,
            out_specs=pl.BlockSpec((BLK, d),
                                   lambda b, be, q0=q0blk: (q0 + b, 0)),
        ),
        out_shape=jax.ShapeDtypeStruct((p, d), jnp.float32),
        input_output_aliases=aliases,
    )(*operands)


def kernel(x, gate_w, fc1_w, fc2_w, fc3_w):
    b, s, d = x.shape
    e = gate_w.shape[0]
    t = b * s
    p = TOP_K * t + e * BLK
    x2d = x.reshape(t, d)

    i0, i1, p0, p1 = _router(x2d, gate_w)

    e_flat = jnp.stack([i0, i1], axis=1).reshape(-1)
    oh = (e_flat[:, None] == jnp.arange(e, dtype=jnp.int32)[None, :]).astype(
        jnp.int32)
    cum = jnp.cumsum(oh, axis=0)
    rank = jnp.sum((cum - oh) * oh, axis=1)
    counts = cum[-1]
    padded = ((counts + BLK - 1) // BLK) * BLK
    starts = jnp.concatenate(
        [jnp.zeros((1,), jnp.int32), jnp.cumsum(padded)[:-1].astype(jnp.int32)])
    pos = (jnp.sum(oh * starts[None, :], axis=1) + rank).astype(jnp.int32)
    pos2 = pos.reshape(t, TOP_K)
    blk_lo = jnp.arange(p // BLK, dtype=jnp.int32)[:, None] * BLK
    block_expert = jnp.sum((blk_lo >= starts[None, :]).astype(jnp.int32),
                           axis=1) - 1
    p0b = jnp.broadcast_to(p0[:, None], (t, 16))
    p1b = jnp.broadcast_to(p1[:, None], (t, 16))

    row_token = _sc_bookkeep(pos, p, t)

    nq = 4
    pq = p // nq
    nblk_q = pq // BLK
    xs_q = [_sc_gather(x2d, row_token[q * pq:(q + 1) * pq], pq)
            for q in range(nq)]
    ys = None
    for q in range(nq):
        be_q = block_expert[q * nblk_q:(q + 1) * nblk_q]
        hs_q = _mlp1(be_q, xs_q[q], fc1_w, fc2_w)
        ys = _mlp2(be_q, hs_q, fc3_w, q * nblk_q, p, ys)
    out2d = _sc_combine(ys, pos2[:, 0], pos2[:, 1], p0b, p1b)
    return out2d.reshape(b, s, d)


# speedup vs baseline: 1.3232x; 1.0565x over previous
import functools

import jax
import jax.numpy as jnp
from jax import lax
from jax.experimental import pallas as pl
from jax.experimental.pallas import tpu as pltpu
from jax.experimental.pallas import tpu_sc as plsc

TOP_K = 2
BLK = 256
TB = 512
NUM_CORES = 2
NUM_SUBCORES = 16
NW = NUM_CORES * NUM_SUBCORES


def _router_body(x_ref, gw_ref, i0_ref, i1_ref, p0_ref, p1_ref):
    xb = x_ref[...]
    gw = gw_ref[...]
    s = lax.dot_general(xb, gw, (((1,), (1,)), ((), ())),
                        preferred_element_type=jnp.float32)
    e = s.shape[1]
    cols = lax.broadcasted_iota(jnp.int32, s.shape, 1)
    m0 = jnp.max(s, axis=1)
    i0 = jnp.min(jnp.where(s == m0[:, None], cols, e), axis=1)
    s2 = jnp.where(cols == i0[:, None], -jnp.inf, s)
    m1 = jnp.max(s2, axis=1)
    i1 = jnp.min(jnp.where(s2 == m1[:, None], cols, e), axis=1)
    p0 = 1.0 / (1.0 + jnp.exp(m1 - m0))
    i0_ref[...] = i0[:, None].astype(jnp.int32)
    i1_ref[...] = i1[:, None].astype(jnp.int32)
    p0_ref[...] = p0[:, None]
    p1_ref[...] = (1.0 - p0)[:, None]


def _router(x2d, gate_w):
    t, d = x2d.shape
    e = gate_w.shape[0]
    outs = pl.pallas_call(
        _router_body,
        grid=(t // TB,),
        in_specs=[
            pl.BlockSpec((TB, d), lambda b: (b, 0)),
            pl.BlockSpec((e, d), lambda b: (0, 0)),
        ],
        out_specs=[pl.BlockSpec((TB, 1), lambda b: (b, 0))] * 4,
        out_shape=[
            jax.ShapeDtypeStruct((t, 1), jnp.int32),
            jax.ShapeDtypeStruct((t, 1), jnp.int32),
            jax.ShapeDtypeStruct((t, 1), jnp.float32),
            jax.ShapeDtypeStruct((t, 1), jnp.float32),
        ],
    )(x2d, gate_w)
    return [o[:, 0] for o in outs]


def _sc_bookkeep(pos, p, t):
    nslots = pos.shape[0]
    spw = nslots // NW
    nj = spw // 128
    mesh = plsc.VectorSubcoreMesh(core_axis_name="c", subcore_axis_name="s")
    pos3 = pos.reshape(NW, nj, 128)
    tok3 = jnp.broadcast_to(
        (jnp.arange(nslots, dtype=jnp.int32) >> 1).reshape(NW, nj, 128, 1),
        (NW, nj, 128, 128))

    @functools.partial(
        pl.kernel,
        out_type=jax.ShapeDtypeStruct((p, 128), jnp.int32),
        mesh=mesh,
        scratch_types=[
            pltpu.VMEM((nj, 128), jnp.int32),
            pltpu.VMEM((nj, 128, 128), jnp.int32),
            pltpu.SemaphoreType.DMA,
            pltpu.SemaphoreType.DMA,
        ],
    )
    def bk_k(pos_hbm, tok_hbm, out_hbm, posw, tokw, sem0, sem1):
        wid = lax.axis_index("s") * NUM_CORES + lax.axis_index("c")
        pltpu.sync_copy(pos_hbm.at[wid], posw)
        pltpu.sync_copy(tok_hbm.at[wid], tokw)
        sems = (sem0, sem1)
        cps = [pltpu.async_copy(tokw.at[j], out_hbm.at[posw.at[j]], sems[j])
               for j in range(nj)]
        for cp in cps:
            cp.wait()

    return bk_k(pos3, tok3)[:, 0]


def _sc_gather(x2d, row_token, p):
    t = x2d.shape[0]
    row = x2d.shape[1:]
    rpw = p // NW
    ch = 8
    nbuf = 6
    n = rpw // ch
    mesh = plsc.VectorSubcoreMesh(core_axis_name="c", subcore_axis_name="s")

    @functools.partial(
        pl.kernel,
        out_type=jax.ShapeDtypeStruct((p,) + row, x2d.dtype),
        mesh=mesh,
        scratch_types=[pltpu.VMEM((rpw,), jnp.int32)]
        + [pltpu.VMEM((ch,) + row, x2d.dtype)] * nbuf
        + [pltpu.SemaphoreType.DMA] * (2 * nbuf),
    )
    def gather_k(x_hbm, tok_hbm, out_hbm, idx_all, *bufs_sems):
        rows = bufs_sems[:nbuf]
        sg = bufs_sems[nbuf:2 * nbuf]
        ss = bufs_sems[2 * nbuf:]
        wid = lax.axis_index("s") * NUM_CORES + lax.axis_index("c")
        base = wid * rpw
        pltpu.sync_copy(tok_hbm.at[pl.ds(base, rpw)], idx_all)

        def clamp(i, carry):
            sl = pl.ds(i * 16, 16)
            v = idx_all[sl]
            idx_all[sl] = jnp.minimum(jnp.maximum(v, 0), t - 1)
            return carry

        lax.fori_loop(0, rpw // 16, clamp, 0)
        gat = [None] * nbuf
        st = [None] * nbuf

        def start(i):
            b = i % nbuf
            gat[b] = pltpu.async_copy(
                x_hbm.at[idx_all.at[pl.ds(i * ch, ch)]], rows[b], sg[b])

        for i in range(min(nbuf, n)):
            start(i)
        for i in range(n):
            b = i % nbuf
            gat[b].wait()
            st[b] = pltpu.async_copy(
                rows[b], out_hbm.at[pl.ds(base + i * ch, ch)], ss[b])
            j = i + nbuf
            if j < n:
                st[b].wait()
                start(j)
        for i in range(max(0, n - nbuf), n):
            st[i % nbuf].wait()

    return gather_k(x2d, row_token)


def _sc_combine(ys, pos0, pos1, p0b, p1b):
    p, d = ys.shape
    t = pos0.shape[0]
    tpw = t // NW
    ch = 8
    nbuf = 2
    n = tpw // ch
    mesh = plsc.VectorSubcoreMesh(core_axis_name="c", subcore_axis_name="s")

    @functools.partial(
        pl.kernel,
        out_type=jax.ShapeDtypeStruct((t, d), jnp.float32),
        mesh=mesh,
        scratch_types=[pltpu.VMEM((tpw,), jnp.int32),
                       pltpu.VMEM((tpw,), jnp.int32),
                       pltpu.VMEM((tpw, 16), jnp.float32),
                       pltpu.VMEM((tpw, 16), jnp.float32)]
        + [pltpu.VMEM((ch, d), jnp.float32)] * (2 * nbuf)
        + [pltpu.SemaphoreType.DMA] * (3 * nbuf),
    )
    def combine_k(y_hbm, p0_hbm, p1_hbm, s0_hbm, s1_hbm, out_hbm,
                  i0_all, i1_all, s0_all, s1_all, *bufs_sems):
        av = bufs_sems[:nbuf]
        bv = bufs_sems[nbuf:2 * nbuf]
        sems = bufs_sems[2 * nbuf:]
        sa = sems[:nbuf]
        sb = sems[nbuf:2 * nbuf]
        ss = sems[2 * nbuf:]
        wid = lax.axis_index("s") * NUM_CORES + lax.axis_index("c")
        base = wid * tpw
        pltpu.sync_copy(p0_hbm.at[pl.ds(base, tpw)], i0_all)
        pltpu.sync_copy(p1_hbm.at[pl.ds(base, tpw)], i1_all)
        pltpu.sync_copy(s0_hbm.at[pl.ds(base, tpw)], s0_all)
        pltpu.sync_copy(s1_hbm.at[pl.ds(base, tpw)], s1_all)
        ga = [None] * nbuf
        gb = [None] * nbuf
        st = [None] * nbuf

        def start(i):
            k = i % nbuf
            sl = pl.ds(i * ch, ch)
            ga[k] = pltpu.async_copy(y_hbm.at[i0_all.at[sl]], av[k], sa[k])
            gb[k] = pltpu.async_copy(y_hbm.at[i1_all.at[sl]], bv[k], sb[k])

        for i in range(min(nbuf, n)):
            start(i)
        for i in range(n):
            k = i % nbuf
            ga[k].wait()
            gb[k].wait()
            a_ref, b_ref = av[k], bv[k]
            s0r = [s0_all[i * ch + r, :] for r in range(ch)]
            s1r = [s1_all[i * ch + r, :] for r in range(ch)]

            def add_col(c, carry, a_ref=a_ref, b_ref=b_ref,
                        s0r=s0r, s1r=s1r):
                sl = pl.ds(c * 16, 16)
                for r in range(ch):
                    a_ref[r, sl] = (s0r[r] * a_ref[r, sl]
                                    + s1r[r] * b_ref[r, sl])
                return carry

            lax.fori_loop(0, d // 16, add_col, 0)
            st[k] = pltpu.async_copy(
                a_ref, out_hbm.at[pl.ds(base + i * ch, ch)], ss[k])
            j = i + nbuf
            if j < n:
                st[k].wait()
                start(j)
        for i in range(max(0, n - nbuf), n):
            st[i % nbuf].wait()

    return combine_k(ys, pos0, pos1, p0b, p1b)


def _mlp1_body(be_ref, xs_ref, w1_ref, w2_ref, h_ref):
    xb = xs_ref[...].astype(jnp.bfloat16)
    w1 = w1_ref[0].astype(jnp.bfloat16)
    w2 = w2_ref[0].astype(jnp.bfloat16)
    dn = (((1,), (1,)), ((), ()))
    h1 = lax.dot_general(xb, w1, dn, preferred_element_type=jnp.float32)
    h2 = lax.dot_general(xb, w2, dn, preferred_element_type=jnp.float32)
    h_ref[...] = (h1 * jax.nn.sigmoid(h1) * h2).astype(jnp.bfloat16)


def _mlp2_body(be_ref, h_ref, w3_ref, out_ref):
    hb = h_ref[...]
    w3 = w3_ref[0].astype(jnp.bfloat16)
    y = lax.dot_general(hb, w3, (((1,), (1,)), ((), ())),
                        preferred_element_type=jnp.float32)
    out_ref[...] = y


def _mlp2_body_alias(be_ref, h_ref, w3_ref, ys_ref, out_ref):
    del ys_ref
    _mlp2_body(be_ref, h_ref, w3_ref, out_ref)


def _mlp1(be_q, xs_q, fc1_w, fc2_w):
    pq, d = xs_q.shape
    e, f, _ = fc1_w.shape
    return pl.pallas_call(
        _mlp1_body,
        grid_spec=pltpu.PrefetchScalarGridSpec(
            num_scalar_prefetch=1,
            grid=(pq // BLK,),
            in_specs=[
                pl.BlockSpec((BLK, d), lambda b, be: (b, 0)),
                pl.BlockSpec((1, f, d), lambda b, be: (be[b], 0, 0)),
                pl.BlockSpec((1, f, d), lambda b, be: (be[b], 0, 0)),
            ],
            out_specs=pl.BlockSpec((BLK, f), lambda b, be: (b, 0)),
        ),
        out_shape=jax.ShapeDtypeStruct((pq, f), jnp.bfloat16),
    )(be_q, xs_q, fc1_w, fc2_w)


def _mlp2(be_q, hs_q, fc3_w, q0blk, p, ys_prev):
    pq, f = hs_q.shape
    d = fc3_w.shape[1]
    in_specs = [
        pl.BlockSpec((BLK, f), lambda b, be: (b, 0)),
        pl.BlockSpec((1, d, f), lambda b, be: (be[b], 0, 0)),
    ]
    operands = [be_q, hs_q, fc3_w]
    body = _mlp2_body
    aliases = {}
    if ys_prev is not None:
        in_specs.append(pl.BlockSpec(memory_space=pltpu.MemorySpace.HBM))
        operands.append(ys_prev)
        body = _mlp2_body_alias
        aliases = {3: 0}
    return pl.pallas_call(
        body,
        grid_spec=pltpu.PrefetchScalarGridSpec(
            num_scalar_prefetch=1,
            grid=(pq // BLK,),
            in_specs=in_specs,
            out_specs=pl.BlockSpec((BLK, d),
                                   lambda b, be, q0=q0blk: (q0 + b, 0)),
        ),
        out_shape=jax.ShapeDtypeStruct((p, d), jnp.float32),
        input_output_aliases=aliases,
    )(*operands)


def kernel(x, gate_w, fc1_w, fc2_w, fc3_w):
    b, s, d = x.shape
    e = gate_w.shape[0]
    t = b * s
    p = TOP_K * t + e * BLK
    x2d = x.reshape(t, d)

    i0, i1, p0, p1 = _router(x2d, gate_w)

    e_flat = jnp.stack([i0, i1], axis=1).reshape(-1)
    oh = (e_flat[:, None] == jnp.arange(e, dtype=jnp.int32)[None, :]).astype(
        jnp.int32)
    cum = jnp.cumsum(oh, axis=0)
    rank = jnp.sum((cum - oh) * oh, axis=1)
    counts = cum[-1]
    padded = ((counts + BLK - 1) // BLK) * BLK
    starts = jnp.concatenate(
        [jnp.zeros((1,), jnp.int32), jnp.cumsum(padded)[:-1].astype(jnp.int32)])
    pos = (jnp.sum(oh * starts[None, :], axis=1) + rank).astype(jnp.int32)
    pos2 = pos.reshape(t, TOP_K)
    blk_lo = jnp.arange(p // BLK, dtype=jnp.int32)[:, None] * BLK
    block_expert = jnp.sum((blk_lo >= starts[None, :]).astype(jnp.int32),
                           axis=1) - 1
    p0b = jnp.broadcast_to(p0[:, None], (t, 16))
    p1b = jnp.broadcast_to(p1[:, None], (t, 16))

    row_token = _sc_bookkeep(pos, p, t)

    nq = 4
    pq = p // nq
    nblk_q = pq // BLK
    xs_q = [_sc_gather(x2d, row_token[q * pq:(q + 1) * pq], pq)
            for q in range(nq)]
    be_q = [block_expert[q * nblk_q:(q + 1) * nblk_q] for q in range(nq)]
    hs_q = [_mlp1(be_q[q], xs_q[q], fc1_w, fc2_w) for q in range(nq)]
    ys = None
    for q in range(nq):
        ys = _mlp2(be_q[q], hs_q[q], fc3_w, q * nblk_q, p, ys)
    out2d = _sc_combine(ys, pos2[:, 0], pos2[:, 1], p0b, p1b)
    return out2d.reshape(b, s, d)


# speedup vs baseline: 1.3565x; 1.0252x over previous
import functools

import jax
import jax.numpy as jnp
from jax import lax
from jax.experimental import pallas as pl
from jax.experimental.pallas import tpu as pltpu
from jax.experimental.pallas import tpu_sc as plsc

TOP_K = 2
BLK = 256
TB = 512
NUM_CORES = 2
NUM_SUBCORES = 16
NW = NUM_CORES * NUM_SUBCORES


def _router_body(x_ref, gw_ref, i0_ref, i1_ref, p0_ref, p1_ref):
    xb = x_ref[...]
    gw = gw_ref[...]
    s = lax.dot_general(xb, gw, (((1,), (1,)), ((), ())),
                        preferred_element_type=jnp.float32)
    e = s.shape[1]
    cols = lax.broadcasted_iota(jnp.int32, s.shape, 1)
    m0 = jnp.max(s, axis=1)
    i0 = jnp.min(jnp.where(s == m0[:, None], cols, e), axis=1)
    s2 = jnp.where(cols == i0[:, None], -jnp.inf, s)
    m1 = jnp.max(s2, axis=1)
    i1 = jnp.min(jnp.where(s2 == m1[:, None], cols, e), axis=1)
    p0 = 1.0 / (1.0 + jnp.exp(m1 - m0))
    i0_ref[...] = i0[:, None].astype(jnp.int32)
    i1_ref[...] = i1[:, None].astype(jnp.int32)
    p0_ref[...] = p0[:, None]
    p1_ref[...] = (1.0 - p0)[:, None]


def _router(x2d, gate_w):
    t, d = x2d.shape
    e = gate_w.shape[0]
    outs = pl.pallas_call(
        _router_body,
        grid=(t // TB,),
        in_specs=[
            pl.BlockSpec((TB, d), lambda b: (b, 0)),
            pl.BlockSpec((e, d), lambda b: (0, 0)),
        ],
        out_specs=[pl.BlockSpec((TB, 1), lambda b: (b, 0))] * 4,
        out_shape=[
            jax.ShapeDtypeStruct((t, 1), jnp.int32),
            jax.ShapeDtypeStruct((t, 1), jnp.int32),
            jax.ShapeDtypeStruct((t, 1), jnp.float32),
            jax.ShapeDtypeStruct((t, 1), jnp.float32),
        ],
    )(x2d, gate_w)
    return [o[:, 0] for o in outs]


def _sc_bookkeep(pos, p, t):
    nslots = pos.shape[0]
    spw = nslots // NW
    nj = spw // 128
    mesh = plsc.VectorSubcoreMesh(core_axis_name="c", subcore_axis_name="s")
    pos3 = pos.reshape(NW, nj, 128)
    tok3 = jnp.broadcast_to(
        (jnp.arange(nslots, dtype=jnp.int32) >> 1).reshape(NW, nj, 128, 1),
        (NW, nj, 128, 128))

    @functools.partial(
        pl.kernel,
        out_type=jax.ShapeDtypeStruct((p, 128), jnp.int32),
        mesh=mesh,
        scratch_types=[
            pltpu.VMEM((nj, 128), jnp.int32),
            pltpu.VMEM((nj, 128, 128), jnp.int32),
            pltpu.SemaphoreType.DMA,
            pltpu.SemaphoreType.DMA,
        ],
    )
    def bk_k(pos_hbm, tok_hbm, out_hbm, posw, tokw, sem0, sem1):
        wid = lax.axis_index("s") * NUM_CORES + lax.axis_index("c")
        pltpu.sync_copy(pos_hbm.at[wid], posw)
        pltpu.sync_copy(tok_hbm.at[wid], tokw)
        sems = (sem0, sem1)
        cps = [pltpu.async_copy(tokw.at[j], out_hbm.at[posw.at[j]], sems[j])
               for j in range(nj)]
        for cp in cps:
            cp.wait()

    return bk_k(pos3, tok3)[:, 0]


def _sc_gather(x2d, row_token, p):
    t = x2d.shape[0]
    row = x2d.shape[1:]
    rpw = p // NW
    ch = 8
    nbuf = 6
    n = rpw // ch
    mesh = plsc.VectorSubcoreMesh(core_axis_name="c", subcore_axis_name="s")

    @functools.partial(
        pl.kernel,
        out_type=jax.ShapeDtypeStruct((p,) + row, x2d.dtype),
        mesh=mesh,
        scratch_types=[pltpu.VMEM((rpw,), jnp.int32)]
        + [pltpu.VMEM((ch,) + row, x2d.dtype)] * nbuf
        + [pltpu.SemaphoreType.DMA] * (2 * nbuf),
    )
    def gather_k(x_hbm, tok_hbm, out_hbm, idx_all, *bufs_sems):
        rows = bufs_sems[:nbuf]
        sg = bufs_sems[nbuf:2 * nbuf]
        ss = bufs_sems[2 * nbuf:]
        wid = lax.axis_index("s") * NUM_CORES + lax.axis_index("c")
        base = wid * rpw
        pltpu.sync_copy(tok_hbm.at[pl.ds(base, rpw)], idx_all)

        def clamp(i, carry):
            sl = pl.ds(i * 16, 16)
            v = idx_all[sl]
            idx_all[sl] = jnp.minimum(jnp.maximum(v, 0), t - 1)
            return carry

        lax.fori_loop(0, rpw // 16, clamp, 0)
        gat = [None] * nbuf
        st = [None] * nbuf

        def start(i):
            b = i % nbuf
            gat[b] = pltpu.async_copy(
                x_hbm.at[idx_all.at[pl.ds(i * ch, ch)]], rows[b], sg[b])

        for i in range(min(nbuf, n)):
            start(i)
        for i in range(n):
            b = i % nbuf
            gat[b].wait()
            st[b] = pltpu.async_copy(
                rows[b], out_hbm.at[pl.ds(base + i * ch, ch)], ss[b])
            j = i + nbuf
            if j < n:
                st[b].wait()
                start(j)
        for i in range(max(0, n - nbuf), n):
            st[i % nbuf].wait()

    return gather_k(x2d, row_token)


def _sc_combine(ys, pos0, pos1, p0b, p1b):
    p, d = ys.shape
    t = pos0.shape[0]
    tpw = t // NW
    ch = 8
    nbuf = 2
    n = tpw // ch
    mesh = plsc.VectorSubcoreMesh(core_axis_name="c", subcore_axis_name="s")

    @functools.partial(
        pl.kernel,
        out_type=jax.ShapeDtypeStruct((t, d), jnp.float32),
        mesh=mesh,
        scratch_types=[pltpu.VMEM((tpw,), jnp.int32),
                       pltpu.VMEM((tpw,), jnp.int32),
                       pltpu.VMEM((tpw, 16), jnp.float32),
                       pltpu.VMEM((tpw, 16), jnp.float32)]
        + [pltpu.VMEM((ch, d), jnp.float32)] * (2 * nbuf)
        + [pltpu.SemaphoreType.DMA] * (3 * nbuf),
    )
    def combine_k(y_hbm, p0_hbm, p1_hbm, s0_hbm, s1_hbm, out_hbm,
                  i0_all, i1_all, s0_all, s1_all, *bufs_sems):
        av = bufs_sems[:nbuf]
        bv = bufs_sems[nbuf:2 * nbuf]
        sems = bufs_sems[2 * nbuf:]
        sa = sems[:nbuf]
        sb = sems[nbuf:2 * nbuf]
        ss = sems[2 * nbuf:]
        wid = lax.axis_index("s") * NUM_CORES + lax.axis_index("c")
        base = wid * tpw
        pltpu.sync_copy(p0_hbm.at[pl.ds(base, tpw)], i0_all)
        pltpu.sync_copy(p1_hbm.at[pl.ds(base, tpw)], i1_all)
        pltpu.sync_copy(s0_hbm.at[pl.ds(base, tpw)], s0_all)
        pltpu.sync_copy(s1_hbm.at[pl.ds(base, tpw)], s1_all)
        ga = [None] * nbuf
        gb = [None] * nbuf
        st = [None] * nbuf

        def start(i):
            k = i % nbuf
            sl = pl.ds(i * ch, ch)
            ga[k] = pltpu.async_copy(y_hbm.at[i0_all.at[sl]], av[k], sa[k])
            gb[k] = pltpu.async_copy(y_hbm.at[i1_all.at[sl]], bv[k], sb[k])

        for i in range(min(nbuf, n)):
            start(i)
        for i in range(n):
            k = i % nbuf
            ga[k].wait()
            gb[k].wait()
            a_ref, b_ref = av[k], bv[k]
            s0r = [s0_all[i * ch + r, :] for r in range(ch)]
            s1r = [s1_all[i * ch + r, :] for r in range(ch)]

            def add_col(c, carry, a_ref=a_ref, b_ref=b_ref,
                        s0r=s0r, s1r=s1r):
                sl = pl.ds(c * 16, 16)
                for r in range(ch):
                    a_ref[r, sl] = (s0r[r] * a_ref[r, sl]
                                    + s1r[r] * b_ref[r, sl])
                return carry

            lax.fori_loop(0, d // 16, add_col, 0)
            st[k] = pltpu.async_copy(
                a_ref, out_hbm.at[pl.ds(base + i * ch, ch)], ss[k])
            j = i + nbuf
            if j < n:
                st[k].wait()
                start(j)
        for i in range(max(0, n - nbuf), n):
            st[i % nbuf].wait()

    return combine_k(ys, pos0, pos1, p0b, p1b)


def _mlp1_body(be_ref, xs_ref, w1_ref, w2_ref, h_ref):
    xb = xs_ref[...].astype(jnp.bfloat16)
    w1 = w1_ref[0].astype(jnp.bfloat16)
    w2 = w2_ref[0].astype(jnp.bfloat16)
    dn = (((1,), (1,)), ((), ()))
    h1 = lax.dot_general(xb, w1, dn, preferred_element_type=jnp.float32)
    h2 = lax.dot_general(xb, w2, dn, preferred_element_type=jnp.float32)
    h_ref[...] = (h1 * jax.nn.sigmoid(h1) * h2).astype(jnp.bfloat16)


def _mlp2_body(be_ref, h_ref, w3_ref, out_ref):
    hb = h_ref[...]
    w3 = w3_ref[0].astype(jnp.bfloat16)
    y = lax.dot_general(hb, w3, (((1,), (1,)), ((), ())),
                        preferred_element_type=jnp.float32)
    out_ref[...] = y


def _mlp2_body_alias(be_ref, h_ref, w3_ref, ys_ref, out_ref):
    del ys_ref
    _mlp2_body(be_ref, h_ref, w3_ref, out_ref)


def _mlp1(be_q, xs_q, fc1_w, fc2_w):
    pq, d = xs_q.shape
    e, f, _ = fc1_w.shape
    return pl.pallas_call(
        _mlp1_body,
        grid_spec=pltpu.PrefetchScalarGridSpec(
            num_scalar_prefetch=1,
            grid=(pq // BLK,),
            in_specs=[
                pl.BlockSpec((BLK, d), lambda b, be: (b, 0)),
                pl.BlockSpec((1, f, d), lambda b, be: (be[b], 0, 0)),
                pl.BlockSpec((1, f, d), lambda b, be: (be[b], 0, 0)),
            ],
            out_specs=pl.BlockSpec((BLK, f), lambda b, be: (b, 0)),
        ),
        out_shape=jax.ShapeDtypeStruct((pq, f), jnp.bfloat16),
    )(be_q, xs_q, fc1_w, fc2_w)


def _mlp2(be_q, hs_q, fc3_w, q0blk, p, ys_prev):
    pq, f = hs_q.shape
    d = fc3_w.shape[1]
    in_specs = [
        pl.BlockSpec((BLK, f), lambda b, be: (b, 0)),
        pl.BlockSpec((1, d, f), lambda b, be: (be[b], 0, 0)),
    ]
    operands = [be_q, hs_q, fc3_w]
    body = _mlp2_body
    aliases = {}
    if ys_prev is not None:
        in_specs.append(pl.BlockSpec(memory_space=pltpu.MemorySpace.HBM))
        operands.append(ys_prev)
        body = _mlp2_body_alias
        aliases = {3: 0}
    return pl.pallas_call(
        body,
        grid_spec=pltpu.PrefetchScalarGridSpec(
            num_scalar_prefetch=1,
            grid=(pq // BLK,),
            in_specs=in_specs,
            out_specs=pl.BlockSpec((BLK, d),
                                   lambda b, be, q0=q0blk: (q0 + b, 0)),
        ),
        out_shape=jax.ShapeDtypeStruct((p, d), jnp.float32),
        input_output_aliases=aliases,
    )(*operands)


def kernel(x, gate_w, fc1_w, fc2_w, fc3_w):
    b, s, d = x.shape
    e = gate_w.shape[0]
    t = b * s
    p = TOP_K * t + e * BLK
    x2d = x.reshape(t, d)

    i0, i1, p0, p1 = _router(x2d, gate_w)

    e_flat = jnp.stack([i0, i1], axis=1).reshape(-1)
    oh = (e_flat[:, None] == jnp.arange(e, dtype=jnp.int32)[None, :]).astype(
        jnp.int32)
    cum = jnp.cumsum(oh, axis=0)
    rank = jnp.sum((cum - oh) * oh, axis=1)
    counts = cum[-1]
    padded = ((counts + BLK - 1) // BLK) * BLK
    starts = jnp.concatenate(
        [jnp.zeros((1,), jnp.int32), jnp.cumsum(padded)[:-1].astype(jnp.int32)])
    pos = (jnp.sum(oh * starts[None, :], axis=1) + rank).astype(jnp.int32)
    pos2 = pos.reshape(t, TOP_K)
    blk_lo = jnp.arange(p // BLK, dtype=jnp.int32)[:, None] * BLK
    block_expert = jnp.sum((blk_lo >= starts[None, :]).astype(jnp.int32),
                           axis=1) - 1
    p0b = jnp.broadcast_to(p0[:, None], (t, 16))
    p1b = jnp.broadcast_to(p1[:, None], (t, 16))

    row_token = _sc_bookkeep(pos, p, t)

    nq = 2
    pq = p // nq
    nblk_q = pq // BLK
    xs_q = [_sc_gather(x2d, row_token[q * pq:(q + 1) * pq], pq)
            for q in range(nq)]
    be_q = [block_expert[q * nblk_q:(q + 1) * nblk_q] for q in range(nq)]
    hs_q = [_mlp1(be_q[q], xs_q[q], fc1_w, fc2_w) for q in range(nq)]
    ys = None
    for q in range(nq):
        ys = _mlp2(be_q[q], hs_q[q], fc3_w, q * nblk_q, p, ys)
    out2d = _sc_combine(ys, pos2[:, 0], pos2[:, 1], p0b, p1b)
    return out2d.reshape(b, s, d)
